# baseline (speedup 1.0000x reference)
import functools

import jax
import jax.numpy as jnp
from jax import lax
from jax.experimental import pallas as pl
from jax.experimental.pallas import tpu as pltpu, tpu_sc as plsc

D_MODEL = 1024
BATCH = 4
SEQ = 4096
NC, NS, NLANE = 2, 16, 16
NW = NC * NS
SC_SEQ = 2048
SPW = SC_SEQ // NW
R = 32
STEPS = SPW // R
VECS = R * D_MODEL // NLANE


@functools.partial(
    pl.kernel,
    out_type=jax.ShapeDtypeStruct((BATCH * SC_SEQ * D_MODEL,), jnp.float32),
    mesh=plsc.VectorSubcoreMesh(core_axis_name="c", subcore_axis_name="s"),
    scratch_types=[
        pltpu.VMEM((R * D_MODEL,), jnp.float32),
        pltpu.VMEM((R * D_MODEL,), jnp.float32),
    ],
)
def _sc_add(x_hbm, pos_hbm, out_hbm, buf, pbuf):
    wid = lax.axis_index("s") * NC + lax.axis_index("c")
    s_base = wid * SPW

    for step in range(STEPS):
        s0 = s_base + step * R
        pltpu.sync_copy(pos_hbm.at[pl.ds(s0 * D_MODEL, R * D_MODEL)], pbuf)
        for b in range(BATCH):
            src0 = (b * SEQ + s0) * D_MODEL
            dst0 = (b * SC_SEQ + s0) * D_MODEL
            pltpu.sync_copy(x_hbm.at[pl.ds(src0, R * D_MODEL)], buf)

            def add_body(i, _):
                off = i * (8 * NLANE)
                for u in range(8):
                    o = off + u * NLANE
                    plsc.addupdate(buf.at[pl.ds(o, NLANE)], pbuf[pl.ds(o, NLANE)])
                return 0

            lax.fori_loop(0, VECS // 8, add_body, 0)
            pltpu.sync_copy(buf, out_hbm.at[pl.ds(dst0, R * D_MODEL)])


_BS = 512


def _add_kernel(x_ref, pos_ref, out_ref):
    out_ref[...] = x_ref[...] + pos_ref[...]


def _tc_add(x, pos_table, bs):
    batch, seq_len, d_model = x.shape
    num_s = seq_len // bs
    return pl.pallas_call(
        _add_kernel,
        grid=(num_s,),
        in_specs=[
            pl.BlockSpec((batch, bs, d_model), lambda i: (0, i, 0)),
            pl.BlockSpec((bs, d_model), lambda i: (i, 0)),
        ],
        out_specs=pl.BlockSpec((batch, bs, d_model), lambda i: (0, i, 0)),
        out_shape=jax.ShapeDtypeStruct(x.shape, x.dtype),
    )(x, pos_table)


def kernel(x, pos_table):
    batch, seq_len, d_model = x.shape
    tc_out = _tc_add(x, pos_table, _BS)
    sc_out = _sc_add(x.reshape(-1), pos_table.reshape(-1))
    patch = sc_out[0:1].reshape(1, 1, 1)
    return lax.dynamic_update_slice(tc_out, patch, (0, 0, 0))


# speedup vs baseline: 3.4092x; 3.4092x over previous
import jax
import jax.numpy as jnp
from jax.experimental import pallas as pl


_BS = 512


def _add_kernel(x_ref, pos_ref, out_ref):
    out_ref[...] = x_ref[...] + pos_ref[...]


def kernel(x, pos_table):
    batch, seq_len, d_model = x.shape
    bs = _BS
    num_s = seq_len // bs

    out = pl.pallas_call(
        _add_kernel,
        grid=(num_s,),
        in_specs=[
            pl.BlockSpec((batch, bs, d_model), lambda i: (0, i, 0)),
            pl.BlockSpec((bs, d_model), lambda i: (i, 0)),
        ],
        out_specs=pl.BlockSpec((batch, bs, d_model), lambda i: (0, i, 0)),
        out_shape=jax.ShapeDtypeStruct(x.shape, x.dtype),
    )(x, pos_table)
    return out
